# Initial kernel scaffold; baseline (speedup 1.0000x reference)
import jax
import jax.numpy as jnp
from jax.experimental import pallas as pl


def kernel(x, time_day, time_week):
    raise NotImplementedError("write your pallas kernel here")


# speedup vs baseline: 3.9200x; 3.9200x over previous
import functools

import jax
import jax.numpy as jnp
from jax import lax
from jax.experimental import pallas as pl
from jax.experimental.pallas import tpu as pltpu
from jax.experimental.pallas import tpu_sc as plsc

_B, _T, _N = 64, 12, 2048
_TIME, _FEAT = 288, 64
_NT = _N * _T
_NC, _NS = 2, 16
_NW = _NC * _NS
_BPW = _B // _NW
_CHUNK = 2048
_NCHUNK = _NT // _CHUNK
_CTAB = _TIME * 7


def _sc_body(x1_hbm, x2_hbm, dt_hbm, wt_hbm, out_hbm,
             ci_v, dt_v, wt_v, ccol_v, dwd_v, dww_v,
             xa_v, xb_v, ob0_v, ob1_v, sem0, sem1):
    wid = lax.axis_index("s") * _NC + lax.axis_index("c")
    lanes = lax.iota(jnp.int32, 16)

    pltpu.sync_copy(dt_hbm, dt_v)
    pltpu.sync_copy(wt_hbm, wt_v)

    def build_dw(i, carry):
        v = lanes + i * 16
        d = lax.div(v, jnp.int32(7))
        dwd_v[pl.ds(i * 16, 16)] = d
        dww_v[pl.ds(i * 16, 16)] = v - d * 7
        return carry
    lax.fori_loop(0, _CTAB // 16, build_dw, 0)

    def build_ci(s, carry):
        b2 = lax.div(s, jnp.int32(_NCHUNK))
        c = s - b2 * _NCHUNK
        bb = wid * _BPW + b2
        pltpu.sync_copy(x1_hbm.at[bb, pl.ds(c * _CHUNK, _CHUNK)], xa_v)
        pltpu.sync_copy(x2_hbm.at[bb, pl.ds(c * _CHUNK, _CHUNK)], xb_v)
        base = b2 * _NT + c * _CHUNK

        def inner(i, icarry):
            o = i * 16
            day = (xa_v[pl.ds(o, 16)] * float(_TIME)).astype(jnp.int32)
            day = jnp.minimum(jnp.maximum(day, 0), _TIME - 1)
            wk = xb_v[pl.ds(o, 16)].astype(jnp.int32)
            wk = jnp.minimum(jnp.maximum(wk, 0), 6)
            ci_v[pl.ds(base + o, 16)] = day * 7 + wk
            return icarry
        lax.fori_loop(0, _CHUNK // 16, inner, 0)
        return carry
    lax.fori_loop(0, _BPW * _NCHUNK, build_ci, 0)

    def f_loop(f, carry):
        fsplat = jnp.full((16,), f, dtype=jnp.int32)

        def ccol_build(i, icarry):
            dv = dwd_v[pl.ds(i * 16, 16)]
            wv = dww_v[pl.ds(i * 16, 16)]
            a = plsc.load_gather(dt_v, [fsplat * _TIME + dv])
            bvec = plsc.load_gather(wt_v, [fsplat * 8 + wv])
            ccol_v[pl.ds(i * 16, 16)] = a + bvec
            return icarry
        lax.fori_loop(0, _CTAB // 16, ccol_build, 0)

        def j_loop(j2, jcarry):
            g = f * _NCHUNK + j2
            for p, (ob, sem) in enumerate(((ob0_v, sem0), (ob1_v, sem1))):
                cc = j2 * 2 + p
                b2 = lax.div(cc, jnp.int32(_NCHUNK))
                c = cc - b2 * _NCHUNK
                bb = wid * _BPW + b2
                dst = out_hbm.at[bb, f, pl.ds(c * _CHUNK, _CHUNK)]

                @pl.when(g >= 1)
                def _wait():
                    pltpu.make_async_copy(ob, dst, sem).wait()

                base = b2 * _NT + c * _CHUNK

                def gat(i, icarry):
                    o = i * 64
                    for u in range(4):
                        civ = ci_v[pl.ds(base + o + u * 16, 16)]
                        ob[pl.ds(o + u * 16, 16)] = plsc.load_gather(
                            ccol_v, [civ])
                    return icarry
                lax.fori_loop(0, _CHUNK // 64, gat, 0)
                pltpu.async_copy(ob, dst, sem)
            return jcarry
        lax.fori_loop(0, _NCHUNK, j_loop, 0)
        return carry
    lax.fori_loop(0, _FEAT, f_loop, 0)

    dummy = out_hbm.at[0, 0, pl.ds(0, _CHUNK)]
    pltpu.make_async_copy(ob0_v, dummy, sem0).wait()
    pltpu.make_async_copy(ob1_v, dummy, sem1).wait()


_sc_call = functools.partial(
    pl.kernel,
    mesh=plsc.VectorSubcoreMesh(core_axis_name="c", subcore_axis_name="s"),
    out_type=jax.ShapeDtypeStruct((_B, _FEAT, _NT), jnp.float32),
    compiler_params=pltpu.CompilerParams(needs_layout_passes=False),
    scratch_types=[
        pltpu.VMEM((_BPW * _NT,), jnp.int32),
        pltpu.VMEM((_FEAT * _TIME,), jnp.float32),
        pltpu.VMEM((_FEAT * 8,), jnp.float32),
        pltpu.VMEM((_CTAB,), jnp.float32),
        pltpu.VMEM((_CTAB,), jnp.int32),
        pltpu.VMEM((_CTAB,), jnp.int32),
        pltpu.VMEM((_CHUNK,), jnp.float32),
        pltpu.VMEM((_CHUNK,), jnp.float32),
        pltpu.VMEM((_CHUNK,), jnp.float32),
        pltpu.VMEM((_CHUNK,), jnp.float32),
        pltpu.SemaphoreType.DMA,
        pltpu.SemaphoreType.DMA,
    ],
)(_sc_body)


@jax.jit
def kernel(x, time_day, time_week):
    x1t = jnp.transpose(x[:, :, :, 1], (0, 2, 1)).reshape(_B, _NT)
    x2t = jnp.transpose(x[:, :, :, 2], (0, 2, 1)).reshape(_B, _NT)
    dt = time_day.T.reshape(-1)
    wt = jnp.pad(time_week.T, ((0, 0), (0, 1))).reshape(-1)
    out = _sc_call(x1t, x2t, dt, wt)
    return out.reshape(_B, _FEAT, _N, _T)


# speedup vs baseline: 8.4032x; 2.1437x over previous
import functools

import jax
import jax.numpy as jnp
from jax import lax
from jax.experimental import pallas as pl
from jax.experimental.pallas import tpu as pltpu
from jax.experimental.pallas import tpu_sc as plsc

_B, _T, _N = 64, 12, 2048
_TIME, _FEAT = 288, 64
_NT = _N * _T
_NC, _NS = 2, 16
_NW = _NC * _NS
_BPW = _B // _NW
_CHUNK = 2048
_NCHUNK = _NT // _CHUNK
_CTAB = _TIME * 7
_XROW = _N * 3


def _sc_body(x_hbm, dt_hbm, wt_hbm, out_hbm,
             ci_v, dt_v, wt_v, ccol_v, dwd_v, dww_v,
             xr_v, ob0_v, ob1_v, sem0, sem1):
    wid = lax.axis_index("s") * _NC + lax.axis_index("c")
    lanes = lax.iota(jnp.int32, 16)

    pltpu.sync_copy(dt_hbm, dt_v)
    pltpu.sync_copy(wt_hbm, wt_v)

    @plsc.parallel_loop(0, _CTAB // 16, unroll=4)
    def _build_dw(i):
        v = lanes + i * 16
        d = lax.div(v, jnp.int32(7))
        dwd_v[pl.ds(i * 16, 16)] = d
        dww_v[pl.ds(i * 16, 16)] = v - d * 7

    for b2 in range(_BPW):
        bb = wid * _BPW + b2

        def t_loop(t, carry):
            pltpu.sync_copy(x_hbm.at[bb, pl.ds(t * _XROW, _XROW)], xr_v)

            @plsc.parallel_loop(0, _N // 16, unroll=4)
            def _build_ci(i):
                nvec = lanes + i * 16
                xi = nvec * 3
                day = (plsc.load_gather(xr_v, [xi + 1])
                       * float(_TIME)).astype(jnp.int32)
                day = jnp.minimum(jnp.maximum(day, 0), _TIME - 1)
                wk = plsc.load_gather(xr_v, [xi + 2]).astype(jnp.int32)
                wk = jnp.minimum(jnp.maximum(wk, 0), 6)
                plsc.store_scatter(ci_v, [b2 * _NT + nvec * _T + t],
                                   day * 7 + wk)
            return carry
        lax.fori_loop(0, _T, t_loop, 0)

    def f_loop(f, carry):
        fsplat = jnp.full((16,), f, dtype=jnp.int32)

        @plsc.parallel_loop(0, _CTAB // 16, unroll=4)
        def _ccol_build(i):
            dv = dwd_v[pl.ds(i * 16, 16)]
            wv = dww_v[pl.ds(i * 16, 16)]
            a = plsc.load_gather(dt_v, [fsplat * _TIME + dv])
            bvec = plsc.load_gather(wt_v, [fsplat * 8 + wv])
            ccol_v[pl.ds(i * 16, 16)] = a + bvec

        def j_loop(j2, jcarry):
            g = f * _NCHUNK + j2
            for p, (ob, sem) in enumerate(((ob0_v, sem0), (ob1_v, sem1))):
                cc = j2 * 2 + p
                b2 = lax.div(cc, jnp.int32(_NCHUNK))
                c = cc - b2 * _NCHUNK
                bb = wid * _BPW + b2
                dst = out_hbm.at[bb, f, pl.ds(c * _CHUNK, _CHUNK)]

                @pl.when(g >= 1)
                def _wait():
                    pltpu.make_async_copy(ob, dst, sem).wait()

                base = b2 * _NT + c * _CHUNK

                @plsc.parallel_loop(0, _CHUNK // 16, unroll=8)
                def _gat(i):
                    o = i * 16
                    civ = ci_v[pl.ds(base + o, 16)]
                    ob[pl.ds(o, 16)] = plsc.load_gather(ccol_v, [civ])

                pltpu.async_copy(ob, dst, sem)
            return jcarry
        lax.fori_loop(0, _NCHUNK, j_loop, 0)
        return carry
    lax.fori_loop(0, _FEAT, f_loop, 0)

    dummy = out_hbm.at[0, 0, pl.ds(0, _CHUNK)]
    pltpu.make_async_copy(ob0_v, dummy, sem0).wait()
    pltpu.make_async_copy(ob1_v, dummy, sem1).wait()


_sc_call = functools.partial(
    pl.kernel,
    mesh=plsc.VectorSubcoreMesh(core_axis_name="c", subcore_axis_name="s"),
    out_type=jax.ShapeDtypeStruct((_B, _FEAT, _NT), jnp.float32),
    compiler_params=pltpu.CompilerParams(needs_layout_passes=False),
    scratch_types=[
        pltpu.VMEM((_BPW * _NT,), jnp.int32),
        pltpu.VMEM((_FEAT * _TIME,), jnp.float32),
        pltpu.VMEM((_FEAT * 8,), jnp.float32),
        pltpu.VMEM((_CTAB,), jnp.float32),
        pltpu.VMEM((_CTAB,), jnp.int32),
        pltpu.VMEM((_CTAB,), jnp.int32),
        pltpu.VMEM((_XROW,), jnp.float32),
        pltpu.VMEM((_CHUNK,), jnp.float32),
        pltpu.VMEM((_CHUNK,), jnp.float32),
        pltpu.SemaphoreType.DMA,
        pltpu.SemaphoreType.DMA,
    ],
)(_sc_body)


@jax.jit
def kernel(x, time_day, time_week):
    xflat = x.reshape(_B, _T * _N * 3)
    dt = time_day.T.reshape(-1)
    wt = jnp.pad(time_week.T, ((0, 0), (0, 1))).reshape(-1)
    out = _sc_call(xflat, dt, wt)
    return out.reshape(_B, _FEAT, _N, _T)


# speedup vs baseline: 31.6644x; 3.7681x over previous
import functools

import jax
import jax.numpy as jnp
from jax import lax
from jax.experimental import pallas as pl
from jax.experimental.pallas import tpu as pltpu
from jax.experimental.pallas import tpu_sc as plsc

_B, _T, _N = 64, 12, 2048
_TIME, _FEAT = 288, 64
_NT = _N * _T
_NC, _NS = 2, 16
_NW = _NC * _NS
_BPW = _B // _NW
_CTAB = _TIME * 7
_FH = 8
_NH = _N // 128


def _sc_body(x_hbm, dt_hbm, wt_hbm, out_hbm,
             ci_v, dt_v, wt_v,
             cc0, cc1, cc2, cc3, cc4, cc5, cc6, cc7,
             xa_v, xb_v, sb0, sb1, sem0, sem1):
    wid = lax.axis_index("s") * _NC + lax.axis_index("c")
    lanes = lax.iota(jnp.int32, 16)
    ccs = (cc0, cc1, cc2, cc3, cc4, cc5, cc6, cc7)

    pltpu.sync_copy(dt_hbm, dt_v)
    pltpu.sync_copy(wt_hbm, wt_v)

    for b2 in range(_BPW):
        bb = wid * _BPW + b2
        bhi = bb // 8
        blo = bb % 8

        def t_loop(t, carry):
            pltpu.sync_copy(x_hbm.at[t, 1, bhi, :, blo, :], xa_v)
            pltpu.sync_copy(x_hbm.at[t, 2, bhi, :, blo, :], xb_v)
            base = b2 * _NT + t * _N

            @plsc.parallel_loop(0, _NH, unroll=2)
            def _build_ci(r):
                for g8 in range(8):
                    day = (xa_v[r, pl.ds(g8 * 16, 16)]
                           * float(_TIME)).astype(jnp.int32)
                    day = jnp.minimum(jnp.maximum(day, 0), _TIME - 1)
                    wk = xb_v[r, pl.ds(g8 * 16, 16)].astype(jnp.int32)
                    wk = jnp.minimum(jnp.maximum(wk, 0), 6)
                    ci_v[pl.ds(base + r * 128 + g8 * 16, 16)] = day * 7 + wk
            return carry
        lax.fori_loop(0, _T, t_loop, 0)

    def fh_loop(f_hi, carry):
        @plsc.parallel_loop(0, _CTAB // 16, unroll=2)
        def _ctab_build(i):
            v = lanes + i * 16
            d = lax.div(v, jnp.int32(7))
            w = v - d * 7
            for fl in range(8):
                f = f_hi * 8 + fl
                a = plsc.load_gather(dt_v, [f * _TIME + d])
                bvec = plsc.load_gather(wt_v, [f * 8 + w])
                ccs[fl][pl.ds(i * 16, 16)] = a + bvec

        def q_loop(q, qcarry):
            g = f_hi * _T + q
            for p, (sb, sem) in enumerate(((sb0, sem0), (sb1, sem1))):
                idx = q * 2 + p
                b2 = lax.div(idx, jnp.int32(_T))
                t = idx - b2 * _T
                bb = wid * _BPW + b2
                dst = out_hbm.at[bb, t, f_hi]

                @pl.when(g >= 1)
                def _wait():
                    pltpu.make_async_copy(sb, dst, sem).wait()

                base = b2 * _NT + t * _N

                @plsc.parallel_loop(0, _NH, unroll=2)
                def _gat(nh):
                    for g8 in range(8):
                        civ = ci_v[pl.ds(base + nh * 128 + g8 * 16, 16)]
                        for fl in range(8):
                            sb[nh, fl, pl.ds(g8 * 16, 16)] = (
                                plsc.load_gather(ccs[fl], [civ]))

                pltpu.async_copy(sb, dst, sem)
            return qcarry
        lax.fori_loop(0, _T, q_loop, 0)
        return carry
    lax.fori_loop(0, _FH, fh_loop, 0)

    dummy = out_hbm.at[0, 0, 0]
    pltpu.make_async_copy(sb0, dummy, sem0).wait()
    pltpu.make_async_copy(sb1, dummy, sem1).wait()


_sc_call = functools.partial(
    pl.kernel,
    mesh=plsc.VectorSubcoreMesh(core_axis_name="c", subcore_axis_name="s"),
    out_type=jax.ShapeDtypeStruct((_B, _T, _FH, _NH, 8, 128), jnp.float32),
    compiler_params=pltpu.CompilerParams(needs_layout_passes=False),
    scratch_types=[
        pltpu.VMEM((_BPW * _NT,), jnp.int32),
        pltpu.VMEM((_FEAT * _TIME,), jnp.float32),
        pltpu.VMEM((_FEAT * 8,), jnp.float32),
    ] + [pltpu.VMEM((_CTAB,), jnp.float32) for _ in range(8)] + [
        pltpu.VMEM((_NH, 128), jnp.float32),
        pltpu.VMEM((_NH, 128), jnp.float32),
        pltpu.VMEM((_NH, 8, 128), jnp.float32),
        pltpu.VMEM((_NH, 8, 128), jnp.float32),
        pltpu.SemaphoreType.DMA,
        pltpu.SemaphoreType.DMA,
    ],
)(_sc_body)


@jax.jit
def kernel(x, time_day, time_week):
    x6 = jnp.transpose(x, (1, 3, 0, 2))
    x6 = x6.reshape(_T, 3, 8, 8, _NH, 128)
    x6 = jnp.transpose(x6, (0, 1, 2, 4, 3, 5))
    dt = time_day.T.reshape(-1)
    wt = jnp.pad(time_week.T, ((0, 0), (0, 1))).reshape(-1)
    out6 = _sc_call(x6, dt, wt)
    out = jnp.transpose(out6, (0, 2, 4, 3, 5, 1))
    return out.reshape(_B, _FEAT, _N, _T)
